# baseline (speedup 1.0000x reference)
import functools

import jax
import jax.numpy as jnp
from jax import lax
from jax.experimental import pallas as pl
from jax.experimental.pallas import tpu as pltpu
from jax.experimental.pallas import tpu_sc as plsc

_T = 0.015
_K = 50
_FT = (745, 317, 444, 556, 673)
_NFT = 5
_H = 778
_HP = 896
_NC16 = _HP // 16
_V = 20000
_VP = 20480
_CH = 4096
_NCH = _VP // _CH
_SUB = 2048
_B = 8
_BIGF = 1.0e9
_BIGF2 = 2.0e9
_BIGI = 2 ** 30


def _dist_body(hT_ref, oT_ref, h2_ref, o2T_ref, mind_ref, acc_ref):
    b = pl.program_id(0)
    j = pl.program_id(1)

    @pl.when(j == 0)
    def _init():
        acc_ref[...] = jnp.full((1, _HP), _BIGF, jnp.float32)

    hT = hT_ref[0]
    oT = oT_ref[0]
    h2 = h2_ref[0]
    lane8 = lax.broadcasted_iota(jnp.int32, (_CH, _B), 1)
    o2 = jnp.sum(jnp.where(lane8 == b, o2T_ref[...], 0.0),
                 axis=1, keepdims=True)

    ms = []
    for s in range(_CH // _SUB):
        gs = lax.dot_general(
            oT[:, s * _SUB:(s + 1) * _SUB], hT, (((0,), (0,)), ((), ())),
            preferred_element_type=jnp.float32)
        o2s = o2[s * _SUB:(s + 1) * _SUB]
        ms.append(jnp.min((h2 + o2s) - gs, axis=0, keepdims=True))
    m = jnp.minimum(ms[0], ms[1]) if len(ms) == 2 else (
        jnp.minimum(jnp.minimum(ms[0], ms[1]), jnp.minimum(ms[2], ms[3])))
    acc_ref[...] = jnp.minimum(acc_ref[...], m)

    @pl.when(j == _NCH - 1)
    def _finish_batch():
        mind_ref[pl.ds(b, 1), :] = jnp.sqrt(
            jnp.maximum(2.0 * acc_ref[...], 0.0))


def _min_dists_tc(hT, oT, h2, o2T):
    return pl.pallas_call(
        _dist_body,
        grid=(_B, _NCH),
        in_specs=[
            pl.BlockSpec((1, 3, _HP), lambda b, j: (b, 0, 0)),
            pl.BlockSpec((1, 3, _CH), lambda b, j: (b, 0, j)),
            pl.BlockSpec((1, 1, _HP), lambda b, j: (b, 0, 0)),
            pl.BlockSpec((_CH, _B), lambda b, j: (j, 0)),
        ],
        out_specs=pl.BlockSpec((_B, _HP), lambda b, j: (0, 0)),
        out_shape=jax.ShapeDtypeStruct((_B, _HP), jnp.float32),
        scratch_shapes=[pltpu.VMEM((1, _HP), jnp.float32)],
    )(hT, oT, h2, o2T)


def _iota16():
    return lax.iota(jnp.int32, 16)


def _vgather(x, idx):
    dnums = lax.GatherDimensionNumbers(
        offset_dims=(), collapsed_slice_dims=(0,), start_index_map=(0,))
    return lax.gather(x, idx[:, None], dnums, (1,),
                      mode=lax.GatherScatterMode.PROMISE_IN_BOUNDS)


def _rot(x, sh):
    return _vgather(x, jnp.remainder(_iota16() + sh, 16))


def _allmin(x):
    for sh in (8, 4, 2, 1):
        x = jnp.minimum(x, _rot(x, sh))
    return x


def _allsum(x):
    for sh in (8, 4, 2, 1):
        x = x + _rot(x, sh)
    return x


def _splat(i):
    return jnp.zeros((16,), jnp.int32) + i


def _ext_i(ref, p, nchunks=4):
    acc = jnp.full((16,), _BIGI, jnp.int32)
    for c in range(nchunks):
        v = ref[pl.ds(c * 16, 16)]
        acc = jnp.minimum(acc, jnp.where(_iota16() + c * 16 == p, v, _BIGI))
    return _allmin(acc)


def _sc_select_body(mind_hbm, out_hbm, orig, arr, vv, uniq, vd, sel, outv,
                    idxs, dus):
    nc = plsc.get_sparse_core_info().num_cores
    wid = lax.axis_index("s") * nc + lax.axis_index("c")

    @pl.when(wid < _B)
    def _work():
        pltpu.sync_copy(mind_hbm.at[pl.ds(wid * _HP, _HP)], orig)

        for c in range(4):
            base = c * 16
            av = jnp.full((16,), _BIGI, jnp.int32)
            for k, f in enumerate(_FT):
                av = jnp.where(_iota16() == (_K + k) - base, f, av)
            arr[pl.ds(base, 16)] = av

        def body1(i, carry):
            lastv, lasti = carry

            def mbody(c, mc):
                m, bi = mc
                v0 = orig[pl.ds(c * 16, 16)]
                v = jnp.where(v0 < _T, v0, _BIGF)
                pos = _iota16() + c * 16
                gt = (v > lastv) | ((v == lastv) & (pos > lasti))
                cand = jnp.where(gt, v, 3.0e9)
                upd = cand < m
                return (jnp.where(upd, cand, m), jnp.where(upd, pos, bi))

            m, bi = lax.fori_loop(
                0, _NC16, mbody,
                (jnp.full((16,), 3.0e9, jnp.float32),
                 jnp.full((16,), _BIGI, jnp.int32)))
            m0 = _allmin(m)
            j0 = _allmin(jnp.where(m == m0, bi, _BIGI))

            iv = _splat(i)
            for c in range(4):
                a = arr[pl.ds(c * 16, 16)]
                arr[pl.ds(c * 16, 16)] = jnp.where(
                    _iota16() + c * 16 == iv, j0, a)
            return (m0, j0)

        lax.fori_loop(0, _K, body1,
                      (jnp.full((16,), -1.0, jnp.float32),
                       jnp.full((16,), -1, jnp.int32)))

        posv = [_iota16() + c * 16 for c in range(4)]
        ndup = jnp.zeros((16,), jnp.int32)
        for c in range(4):
            vv[pl.ds(c * 16, 16)] = arr[pl.ds(c * 16, 16)]
        for k, f in enumerate(_FT):
            cnt = jnp.zeros((16,), jnp.int32)
            for c in range(4):
                av = arr[pl.ds(c * 16, 16)]
                cnt = cnt + jnp.where((av == f) & (posv[c] < _K), 1, 0)
            isdup = _allsum(cnt) > 0
            ndup = ndup + jnp.where(isdup, 1, 0)
            v3 = vv[pl.ds(48, 16)]
            vv[pl.ds(48, 16)] = jnp.where(
                (_iota16() == (_K + k) - 48) & isdup, _BIGI, v3)
        n_u = (_K + _NFT) - ndup

        for c in range(4):
            uniq[pl.ds(c * 16, 16)] = jnp.full((16,), -1, jnp.int32)

        def body2(p, _):
            vp = _ext_i(vv, _splat(p))
            rank = jnp.zeros((16,), jnp.int32)
            for c in range(4):
                cv = vv[pl.ds(c * 16, 16)]
                rank = rank + jnp.where(cv < vp, 1, 0)
            rank = _allsum(rank)
            for c in range(4):
                u = uniq[pl.ds(c * 16, 16)]
                uniq[pl.ds(c * 16, 16)] = jnp.where(
                    (posv[c] == rank) & (vp < _BIGI), vp, u)
            return 0
        lax.fori_loop(0, _K + _NFT, body2, 0)

        for c in range(4):
            uc = uniq[pl.ds(c * 16, 16)]
            idxs[pl.ds(c * 16, 16)] = jnp.maximum(uc, 0) + wid * _HP
        pltpu.sync_copy(mind_hbm.at[idxs], dus)
        for c in range(4):
            uc = uniq[pl.ds(c * 16, 16)]
            ok = (posv[c] < n_u) & (uc >= 0)
            vd[pl.ds(c * 16, 16)] = jnp.where(ok, dus[pl.ds(c * 16, 16)],
                                              _BIGF)

        def body4(i, carry):
            lastv, lasti = carry
            m = jnp.full((16,), 3.0e9, jnp.float32)
            bi = jnp.full((16,), _BIGI, jnp.int32)
            for c in range(4):
                v = vd[pl.ds(c * 16, 16)]
                gt = (v > lastv) | ((v == lastv) & (posv[c] > lasti))
                cand = jnp.where(gt, v, 3.0e9)
                upd = cand < m
                m = jnp.where(upd, cand, m)
                bi = jnp.where(upd, posv[c], bi)
            m0 = _allmin(m)
            pj = _allmin(jnp.where(m == m0, bi, _BIGI))

            uj = _ext_i(uniq, pj)
            iv = _splat(i)
            for c in range(4):
                s = sel[pl.ds(c * 16, 16)]
                sel[pl.ds(c * 16, 16)] = jnp.where(posv[c] == iv, uj, s)
            return (m0, pj)

        lax.fori_loop(0, _K, body4,
                      (jnp.full((16,), -1.0, jnp.float32),
                       jnp.full((16,), -1, jnp.int32)))

        trunc = n_u > _K
        for c in range(8):
            outv[pl.ds(c * 16, 16)] = jnp.full((16,), -1, jnp.int32)
        for c in range(4):
            outv[pl.ds(c * 16, 16)] = jnp.where(
                trunc, sel[pl.ds(c * 16, 16)], uniq[pl.ds(c * 16, 16)])
        pltpu.sync_copy(outv, out_hbm.at[wid])


def _select_sc(mind):
    mesh = plsc.VectorSubcoreMesh(core_axis_name="c", subcore_axis_name="s")
    f = functools.partial(
        pl.kernel, mesh=mesh,
        out_type=jax.ShapeDtypeStruct((_B, 128), jnp.int32),
        scratch_types=[
            pltpu.VMEM((_HP,), jnp.float32),
            pltpu.VMEM((128,), jnp.int32),
            pltpu.VMEM((128,), jnp.int32),
            pltpu.VMEM((128,), jnp.int32),
            pltpu.VMEM((128,), jnp.float32),
            pltpu.VMEM((128,), jnp.int32),
            pltpu.VMEM((128,), jnp.int32),
            pltpu.VMEM((64,), jnp.int32),
            pltpu.VMEM((64,), jnp.float32),
        ],
    )(_sc_select_body)
    return f(mind.reshape(_B * _HP))


def kernel(hand_verts, obj_verts, iteration):
    del iteration
    hT = jnp.pad(jnp.transpose(hand_verts, (0, 2, 1)),
                 ((0, 0), (0, 0), (0, _HP - _H)),
                 constant_values=-1000.0)
    oT = jnp.pad(jnp.transpose(obj_verts, (0, 2, 1)),
                 ((0, 0), (0, 0), (0, _VP - _V)),
                 constant_values=1000.0)
    h2 = (0.5 * jnp.pad(jnp.sum(hand_verts * hand_verts, axis=-1),
                        ((0, 0), (0, _HP - _H)),
                        constant_values=_BIGF)).reshape(_B, 1, _HP)
    o2T = jnp.transpose(0.5 * jnp.sum(oT * oT, axis=1), (1, 0))

    mind = _min_dists_tc(hT, oT, h2, o2T)
    out = _select_sc(mind)
    return tuple(out[b, :_K] for b in range(_B))


# speedup vs baseline: 1.0057x; 1.0057x over previous
import functools

import jax
import jax.numpy as jnp
from jax import lax
from jax.experimental import pallas as pl
from jax.experimental.pallas import tpu as pltpu
from jax.experimental.pallas import tpu_sc as plsc

_T = 0.015
_K = 50
_FT = (745, 317, 444, 556, 673)
_NFT = 5
_H = 778
_HP = 896
_NC16 = _HP // 16
_V = 20000
_VP = 20480
_CH = 4096
_NCH = _VP // _CH
_SUB = 512
_B = 8
_BIGF = 1.0e9
_BIGF2 = 2.0e9
_BIGI = 2 ** 30


def _dist_body(hT_ref, oT_ref, h2_ref, o2T_ref, mind_ref, acc_ref):
    b = pl.program_id(0)
    j = pl.program_id(1)

    @pl.when(j == 0)
    def _init():
        acc_ref[...] = jnp.full((1, _HP), _BIGF, jnp.float32)

    hT = hT_ref[0]
    oT = oT_ref[0]
    h2 = h2_ref[0]
    lane8 = lax.broadcasted_iota(jnp.int32, (_CH, _B), 1)
    o2 = jnp.sum(jnp.where(lane8 == b, o2T_ref[...], 0.0),
                 axis=1, keepdims=True)

    ms = []
    for s in range(_CH // _SUB):
        gs = lax.dot_general(
            oT[:, s * _SUB:(s + 1) * _SUB], hT, (((0,), (0,)), ((), ())),
            preferred_element_type=jnp.float32)
        o2s = o2[s * _SUB:(s + 1) * _SUB]
        ms.append(jnp.min((h2 + o2s) - gs, axis=0, keepdims=True))
    while len(ms) > 1:
        ms = [jnp.minimum(ms[2 * i], ms[2 * i + 1])
              for i in range(len(ms) // 2)]
    m = ms[0]
    acc_ref[...] = jnp.minimum(acc_ref[...], m)

    @pl.when(j == _NCH - 1)
    def _finish_batch():
        mind_ref[pl.ds(b, 1), :] = jnp.sqrt(
            jnp.maximum(2.0 * acc_ref[...], 0.0))


def _min_dists_tc(hT, oT, h2, o2T):
    return pl.pallas_call(
        _dist_body,
        grid=(_B, _NCH),
        in_specs=[
            pl.BlockSpec((1, 3, _HP), lambda b, j: (b, 0, 0)),
            pl.BlockSpec((1, 3, _CH), lambda b, j: (b, 0, j)),
            pl.BlockSpec((1, 1, _HP), lambda b, j: (b, 0, 0)),
            pl.BlockSpec((_CH, _B), lambda b, j: (j, 0)),
        ],
        out_specs=pl.BlockSpec((_B, _HP), lambda b, j: (0, 0)),
        out_shape=jax.ShapeDtypeStruct((_B, _HP), jnp.float32),
        scratch_shapes=[pltpu.VMEM((1, _HP), jnp.float32)],
    )(hT, oT, h2, o2T)


def _iota16():
    return lax.iota(jnp.int32, 16)


def _vgather(x, idx):
    dnums = lax.GatherDimensionNumbers(
        offset_dims=(), collapsed_slice_dims=(0,), start_index_map=(0,))
    return lax.gather(x, idx[:, None], dnums, (1,),
                      mode=lax.GatherScatterMode.PROMISE_IN_BOUNDS)


def _rot(x, sh):
    return _vgather(x, jnp.remainder(_iota16() + sh, 16))


def _allmin(x):
    for sh in (8, 4, 2, 1):
        x = jnp.minimum(x, _rot(x, sh))
    return x


def _allsum(x):
    for sh in (8, 4, 2, 1):
        x = x + _rot(x, sh)
    return x


def _splat(i):
    return jnp.zeros((16,), jnp.int32) + i


def _ext_i(ref, p, nchunks=4):
    acc = jnp.full((16,), _BIGI, jnp.int32)
    for c in range(nchunks):
        v = ref[pl.ds(c * 16, 16)]
        acc = jnp.minimum(acc, jnp.where(_iota16() + c * 16 == p, v, _BIGI))
    return _allmin(acc)


def _sc_select_body(mind_hbm, out_hbm, orig, arr, vv, uniq, vd, sel, outv,
                    idxs, dus):
    nc = plsc.get_sparse_core_info().num_cores
    wid = lax.axis_index("s") * nc + lax.axis_index("c")

    @pl.when(wid < _B)
    def _work():
        pltpu.sync_copy(mind_hbm.at[pl.ds(wid * _HP, _HP)], orig)

        for c in range(4):
            base = c * 16
            av = jnp.full((16,), _BIGI, jnp.int32)
            for k, f in enumerate(_FT):
                av = jnp.where(_iota16() == (_K + k) - base, f, av)
            arr[pl.ds(base, 16)] = av

        def body1(i, carry):
            lastv, lasti = carry

            def mbody(c, mc):
                m, bi = mc
                v0 = orig[pl.ds(c * 16, 16)]
                v = jnp.where(v0 < _T, v0, _BIGF)
                pos = _iota16() + c * 16
                gt = (v > lastv) | ((v == lastv) & (pos > lasti))
                cand = jnp.where(gt, v, 3.0e9)
                upd = cand < m
                return (jnp.where(upd, cand, m), jnp.where(upd, pos, bi))

            m, bi = lax.fori_loop(
                0, _NC16, mbody,
                (jnp.full((16,), 3.0e9, jnp.float32),
                 jnp.full((16,), _BIGI, jnp.int32)))
            m0 = _allmin(m)
            j0 = _allmin(jnp.where(m == m0, bi, _BIGI))

            iv = _splat(i)
            for c in range(4):
                a = arr[pl.ds(c * 16, 16)]
                arr[pl.ds(c * 16, 16)] = jnp.where(
                    _iota16() + c * 16 == iv, j0, a)
            return (m0, j0)

        lax.fori_loop(0, _K, body1,
                      (jnp.full((16,), -1.0, jnp.float32),
                       jnp.full((16,), -1, jnp.int32)))

        posv = [_iota16() + c * 16 for c in range(4)]
        ndup = jnp.zeros((16,), jnp.int32)
        for c in range(4):
            vv[pl.ds(c * 16, 16)] = arr[pl.ds(c * 16, 16)]
        for k, f in enumerate(_FT):
            cnt = jnp.zeros((16,), jnp.int32)
            for c in range(4):
                av = arr[pl.ds(c * 16, 16)]
                cnt = cnt + jnp.where((av == f) & (posv[c] < _K), 1, 0)
            isdup = _allsum(cnt) > 0
            ndup = ndup + jnp.where(isdup, 1, 0)
            v3 = vv[pl.ds(48, 16)]
            vv[pl.ds(48, 16)] = jnp.where(
                (_iota16() == (_K + k) - 48) & isdup, _BIGI, v3)
        n_u = (_K + _NFT) - ndup

        for c in range(4):
            uniq[pl.ds(c * 16, 16)] = jnp.full((16,), -1, jnp.int32)

        def body2(p, _):
            vp = _ext_i(vv, _splat(p))
            rank = jnp.zeros((16,), jnp.int32)
            for c in range(4):
                cv = vv[pl.ds(c * 16, 16)]
                rank = rank + jnp.where(cv < vp, 1, 0)
            rank = _allsum(rank)
            for c in range(4):
                u = uniq[pl.ds(c * 16, 16)]
                uniq[pl.ds(c * 16, 16)] = jnp.where(
                    (posv[c] == rank) & (vp < _BIGI), vp, u)
            return 0
        lax.fori_loop(0, _K + _NFT, body2, 0)

        for c in range(4):
            uc = uniq[pl.ds(c * 16, 16)]
            idxs[pl.ds(c * 16, 16)] = jnp.maximum(uc, 0) + wid * _HP
        pltpu.sync_copy(mind_hbm.at[idxs], dus)
        for c in range(4):
            uc = uniq[pl.ds(c * 16, 16)]
            ok = (posv[c] < n_u) & (uc >= 0)
            vd[pl.ds(c * 16, 16)] = jnp.where(ok, dus[pl.ds(c * 16, 16)],
                                              _BIGF)

        def body4(i, carry):
            lastv, lasti = carry
            m = jnp.full((16,), 3.0e9, jnp.float32)
            bi = jnp.full((16,), _BIGI, jnp.int32)
            for c in range(4):
                v = vd[pl.ds(c * 16, 16)]
                gt = (v > lastv) | ((v == lastv) & (posv[c] > lasti))
                cand = jnp.where(gt, v, 3.0e9)
                upd = cand < m
                m = jnp.where(upd, cand, m)
                bi = jnp.where(upd, posv[c], bi)
            m0 = _allmin(m)
            pj = _allmin(jnp.where(m == m0, bi, _BIGI))

            uj = _ext_i(uniq, pj)
            iv = _splat(i)
            for c in range(4):
                s = sel[pl.ds(c * 16, 16)]
                sel[pl.ds(c * 16, 16)] = jnp.where(posv[c] == iv, uj, s)
            return (m0, pj)

        lax.fori_loop(0, _K, body4,
                      (jnp.full((16,), -1.0, jnp.float32),
                       jnp.full((16,), -1, jnp.int32)))

        trunc = n_u > _K
        for c in range(8):
            outv[pl.ds(c * 16, 16)] = jnp.full((16,), -1, jnp.int32)
        for c in range(4):
            outv[pl.ds(c * 16, 16)] = jnp.where(
                trunc, sel[pl.ds(c * 16, 16)], uniq[pl.ds(c * 16, 16)])
        pltpu.sync_copy(outv, out_hbm.at[wid])


def _select_sc(mind):
    mesh = plsc.VectorSubcoreMesh(core_axis_name="c", subcore_axis_name="s")
    f = functools.partial(
        pl.kernel, mesh=mesh,
        out_type=jax.ShapeDtypeStruct((_B, 128), jnp.int32),
        scratch_types=[
            pltpu.VMEM((_HP,), jnp.float32),
            pltpu.VMEM((128,), jnp.int32),
            pltpu.VMEM((128,), jnp.int32),
            pltpu.VMEM((128,), jnp.int32),
            pltpu.VMEM((128,), jnp.float32),
            pltpu.VMEM((128,), jnp.int32),
            pltpu.VMEM((128,), jnp.int32),
            pltpu.VMEM((64,), jnp.int32),
            pltpu.VMEM((64,), jnp.float32),
        ],
    )(_sc_select_body)
    return f(mind.reshape(_B * _HP))


def kernel(hand_verts, obj_verts, iteration):
    del iteration
    hT = jnp.pad(jnp.transpose(hand_verts, (0, 2, 1)),
                 ((0, 0), (0, 0), (0, _HP - _H)),
                 constant_values=-1000.0)
    oT = jnp.pad(jnp.transpose(obj_verts, (0, 2, 1)),
                 ((0, 0), (0, 0), (0, _VP - _V)),
                 constant_values=1000.0)
    h2 = (0.5 * jnp.pad(jnp.sum(hand_verts * hand_verts, axis=-1),
                        ((0, 0), (0, _HP - _H)),
                        constant_values=_BIGF)).reshape(_B, 1, _HP)
    o2T = jnp.transpose(0.5 * jnp.sum(oT * oT, axis=1), (1, 0))

    mind = _min_dists_tc(hT, oT, h2, o2T)
    out = _select_sc(mind)
    return tuple(out[b, :_K] for b in range(_B))


# speedup vs baseline: 1.1767x; 1.1700x over previous
import functools

import jax
import jax.numpy as jnp
from jax import lax
from jax.experimental import pallas as pl
from jax.experimental.pallas import tpu as pltpu
from jax.experimental.pallas import tpu_sc as plsc

_T = 0.015
_K = 50
_FT = (745, 317, 444, 556, 673)
_NFT = 5
_H = 778
_HP = 896
_NC16 = _HP // 16
_V = 20000
_VP = 20480
_CH = 10240
_NCH = _VP // _CH
_SUB = 1024
_B = 8
_BIGF = 1.0e9
_BIGF2 = 2.0e9
_BIGI = 2 ** 30


def _dist_body(hT_ref, oT_ref, h2_ref, o2T_ref, mind_ref, acc_ref):
    b = pl.program_id(0)
    j = pl.program_id(1)

    @pl.when(j == 0)
    def _init():
        acc_ref[...] = jnp.full((1, _HP), _BIGF, jnp.float32)

    hT = hT_ref[0]
    oT = oT_ref[0]
    h2 = h2_ref[0]
    lane8 = lax.broadcasted_iota(jnp.int32, (_CH, _B), 1)
    o2 = jnp.sum(jnp.where(lane8 == b, o2T_ref[...], 0.0),
                 axis=1, keepdims=True)

    ms = []
    for s in range(_CH // _SUB):
        gs = lax.dot_general(
            oT[:, s * _SUB:(s + 1) * _SUB], hT, (((0,), (0,)), ((), ())),
            preferred_element_type=jnp.float32)
        o2s = o2[s * _SUB:(s + 1) * _SUB]
        ms.append(jnp.min((h2 + o2s) - gs, axis=0, keepdims=True))
    while len(ms) > 1:
        ms = [jnp.minimum(ms[2 * i], ms[2 * i + 1])
              for i in range(len(ms) // 2)]
    m = ms[0]
    acc_ref[...] = jnp.minimum(acc_ref[...], m)

    @pl.when(j == _NCH - 1)
    def _finish_batch():
        mind_ref[pl.ds(b, 1), :] = jnp.sqrt(
            jnp.maximum(2.0 * acc_ref[...], 0.0))


def _min_dists_tc(hT, oT, h2, o2T):
    return pl.pallas_call(
        _dist_body,
        grid=(_B, _NCH),
        in_specs=[
            pl.BlockSpec((1, 3, _HP), lambda b, j: (b, 0, 0)),
            pl.BlockSpec((1, 3, _CH), lambda b, j: (b, 0, j)),
            pl.BlockSpec((1, 1, _HP), lambda b, j: (b, 0, 0)),
            pl.BlockSpec((_CH, _B), lambda b, j: (j, 0)),
        ],
        out_specs=pl.BlockSpec((_B, _HP), lambda b, j: (0, 0)),
        out_shape=jax.ShapeDtypeStruct((_B, _HP), jnp.float32),
        scratch_shapes=[pltpu.VMEM((1, _HP), jnp.float32)],
    )(hT, oT, h2, o2T)


def _iota16():
    return lax.iota(jnp.int32, 16)


def _vgather(x, idx):
    dnums = lax.GatherDimensionNumbers(
        offset_dims=(), collapsed_slice_dims=(0,), start_index_map=(0,))
    return lax.gather(x, idx[:, None], dnums, (1,),
                      mode=lax.GatherScatterMode.PROMISE_IN_BOUNDS)


def _rot(x, sh):
    return _vgather(x, jnp.remainder(_iota16() + sh, 16))


def _allmin(x):
    for sh in (8, 4, 2, 1):
        x = jnp.minimum(x, _rot(x, sh))
    return x


def _allsum(x):
    for sh in (8, 4, 2, 1):
        x = x + _rot(x, sh)
    return x


def _splat(i):
    return jnp.zeros((16,), jnp.int32) + i


def _ext_i(ref, p, nchunks=4):
    acc = jnp.full((16,), _BIGI, jnp.int32)
    for c in range(nchunks):
        v = ref[pl.ds(c * 16, 16)]
        acc = jnp.minimum(acc, jnp.where(_iota16() + c * 16 == p, v, _BIGI))
    return _allmin(acc)


def _sc_select_body(mind_hbm, out_hbm, orig, arr, vv, uniq, vd, sel, outv,
                    idxs, dus):
    nc = plsc.get_sparse_core_info().num_cores
    wid = lax.axis_index("s") * nc + lax.axis_index("c")

    @pl.when(wid < _B)
    def _work():
        pltpu.sync_copy(mind_hbm.at[pl.ds(wid * _HP, _HP)], orig)

        for c in range(4):
            base = c * 16
            av = jnp.full((16,), _BIGI, jnp.int32)
            for k, f in enumerate(_FT):
                av = jnp.where(_iota16() == (_K + k) - base, f, av)
            arr[pl.ds(base, 16)] = av

        def body1(i, carry):
            lastv, lasti = carry

            def mbody(c, mc):
                m, bi = mc
                v0 = orig[pl.ds(c * 16, 16)]
                v = jnp.where(v0 < _T, v0, _BIGF)
                pos = _iota16() + c * 16
                gt = (v > lastv) | ((v == lastv) & (pos > lasti))
                cand = jnp.where(gt, v, 3.0e9)
                upd = cand < m
                return (jnp.where(upd, cand, m), jnp.where(upd, pos, bi))

            m, bi = lax.fori_loop(
                0, _NC16, mbody,
                (jnp.full((16,), 3.0e9, jnp.float32),
                 jnp.full((16,), _BIGI, jnp.int32)))
            m0 = _allmin(m)
            j0 = _allmin(jnp.where(m == m0, bi, _BIGI))

            iv = _splat(i)
            for c in range(4):
                a = arr[pl.ds(c * 16, 16)]
                arr[pl.ds(c * 16, 16)] = jnp.where(
                    _iota16() + c * 16 == iv, j0, a)
            return (m0, j0)

        lax.fori_loop(0, _K, body1,
                      (jnp.full((16,), -1.0, jnp.float32),
                       jnp.full((16,), -1, jnp.int32)))

        posv = [_iota16() + c * 16 for c in range(4)]
        ndup = jnp.zeros((16,), jnp.int32)
        for c in range(4):
            vv[pl.ds(c * 16, 16)] = arr[pl.ds(c * 16, 16)]
        for k, f in enumerate(_FT):
            cnt = jnp.zeros((16,), jnp.int32)
            for c in range(4):
                av = arr[pl.ds(c * 16, 16)]
                cnt = cnt + jnp.where((av == f) & (posv[c] < _K), 1, 0)
            isdup = _allsum(cnt) > 0
            ndup = ndup + jnp.where(isdup, 1, 0)
            v3 = vv[pl.ds(48, 16)]
            vv[pl.ds(48, 16)] = jnp.where(
                (_iota16() == (_K + k) - 48) & isdup, _BIGI, v3)
        n_u = (_K + _NFT) - ndup

        for c in range(4):
            uniq[pl.ds(c * 16, 16)] = jnp.full((16,), -1, jnp.int32)

        def body2(p, _):
            vp = _ext_i(vv, _splat(p))
            rank = jnp.zeros((16,), jnp.int32)
            for c in range(4):
                cv = vv[pl.ds(c * 16, 16)]
                rank = rank + jnp.where(cv < vp, 1, 0)
            rank = _allsum(rank)
            for c in range(4):
                u = uniq[pl.ds(c * 16, 16)]
                uniq[pl.ds(c * 16, 16)] = jnp.where(
                    (posv[c] == rank) & (vp < _BIGI), vp, u)
            return 0
        lax.fori_loop(0, _K + _NFT, body2, 0)

        for c in range(4):
            uc = uniq[pl.ds(c * 16, 16)]
            idxs[pl.ds(c * 16, 16)] = jnp.maximum(uc, 0) + wid * _HP
        pltpu.sync_copy(mind_hbm.at[idxs], dus)
        for c in range(4):
            uc = uniq[pl.ds(c * 16, 16)]
            ok = (posv[c] < n_u) & (uc >= 0)
            vd[pl.ds(c * 16, 16)] = jnp.where(ok, dus[pl.ds(c * 16, 16)],
                                              _BIGF)

        def body4(i, carry):
            lastv, lasti = carry
            m = jnp.full((16,), 3.0e9, jnp.float32)
            bi = jnp.full((16,), _BIGI, jnp.int32)
            for c in range(4):
                v = vd[pl.ds(c * 16, 16)]
                gt = (v > lastv) | ((v == lastv) & (posv[c] > lasti))
                cand = jnp.where(gt, v, 3.0e9)
                upd = cand < m
                m = jnp.where(upd, cand, m)
                bi = jnp.where(upd, posv[c], bi)
            m0 = _allmin(m)
            pj = _allmin(jnp.where(m == m0, bi, _BIGI))

            uj = _ext_i(uniq, pj)
            iv = _splat(i)
            for c in range(4):
                s = sel[pl.ds(c * 16, 16)]
                sel[pl.ds(c * 16, 16)] = jnp.where(posv[c] == iv, uj, s)
            return (m0, pj)

        lax.fori_loop(0, _K, body4,
                      (jnp.full((16,), -1.0, jnp.float32),
                       jnp.full((16,), -1, jnp.int32)))

        trunc = n_u > _K
        for c in range(8):
            outv[pl.ds(c * 16, 16)] = jnp.full((16,), -1, jnp.int32)
        for c in range(4):
            outv[pl.ds(c * 16, 16)] = jnp.where(
                trunc, sel[pl.ds(c * 16, 16)], uniq[pl.ds(c * 16, 16)])
        pltpu.sync_copy(outv, out_hbm.at[wid])


def _select_sc(mind):
    mesh = plsc.VectorSubcoreMesh(core_axis_name="c", subcore_axis_name="s")
    f = functools.partial(
        pl.kernel, mesh=mesh,
        out_type=jax.ShapeDtypeStruct((_B, 128), jnp.int32),
        scratch_types=[
            pltpu.VMEM((_HP,), jnp.float32),
            pltpu.VMEM((128,), jnp.int32),
            pltpu.VMEM((128,), jnp.int32),
            pltpu.VMEM((128,), jnp.int32),
            pltpu.VMEM((128,), jnp.float32),
            pltpu.VMEM((128,), jnp.int32),
            pltpu.VMEM((128,), jnp.int32),
            pltpu.VMEM((64,), jnp.int32),
            pltpu.VMEM((64,), jnp.float32),
        ],
    )(_sc_select_body)
    return f(mind.reshape(_B * _HP))


def kernel(hand_verts, obj_verts, iteration):
    del iteration
    hT = jnp.pad(jnp.transpose(hand_verts, (0, 2, 1)),
                 ((0, 0), (0, 0), (0, _HP - _H)),
                 constant_values=-1000.0)
    oT = jnp.pad(jnp.transpose(obj_verts, (0, 2, 1)),
                 ((0, 0), (0, 0), (0, _VP - _V)),
                 constant_values=1000.0)
    h2 = (0.5 * jnp.pad(jnp.sum(hand_verts * hand_verts, axis=-1),
                        ((0, 0), (0, _HP - _H)),
                        constant_values=_BIGF)).reshape(_B, 1, _HP)
    o2T = jnp.transpose(0.5 * jnp.sum(oT * oT, axis=1), (1, 0))

    mind = _min_dists_tc(hT, oT, h2, o2T)
    out = _select_sc(mind)
    return tuple(out[b, :_K] for b in range(_B))
